# baseline (speedup 1.0000x reference)
import functools

import jax
import jax.numpy as jnp
from jax import lax
from jax.experimental import pallas as pl
from jax.experimental.pallas import tpu as pltpu
from jax.experimental.pallas import tpu_sc as plsc

VOCAB = 100000
HIDDEN = 32
BATCH = 1024


@functools.cache
def _make_sc_gather():
    info = plsc.get_sparse_core_info()
    nc, ns = info.num_cores, info.num_subcores
    b_per_w = BATCH // (nc * ns)
    mesh = plsc.VectorSubcoreMesh(core_axis_name="c", subcore_axis_name="s")

    @functools.partial(
        pl.kernel,
        mesh=mesh,
        out_type=jax.ShapeDtypeStruct((BATCH, HIDDEN), jnp.float32),
        scratch_types=[
            pltpu.VMEM((b_per_w,), jnp.int32),
            pltpu.VMEM((b_per_w, HIDDEN), jnp.float32),
            pltpu.SemaphoreType.DMA,
        ],
        compiler_params=pltpu.CompilerParams(use_tc_tiling_on_sc=False),
    )
    def _sc_gather(idx_hbm, table_hbm, out_hbm, idx_v, rows_v, sem):
        wid = lax.axis_index("s") * nc + lax.axis_index("c")
        base = wid * b_per_w
        pltpu.sync_copy(idx_hbm.at[pl.ds(base, b_per_w)], idx_v)
        pltpu.async_copy(table_hbm.at[idx_v], rows_v, sem).wait()
        pltpu.sync_copy(rows_v, out_hbm.at[pl.ds(base, b_per_w)])

    return _sc_gather


_RB = 128
_QW = 24576
_NQ = 4
_NS = BATCH // _RB
_KSUB = _QW // 2048
_MAIN = _NQ * _QW
_TAIL = VOCAB - _MAIN


def _proj_body(h_ref, w_ref, wt_tail_ref, b_ref, bt_ref, o_hbm,
               scratch, tail_buf, sems, tail_sem):
    i = pl.program_id(0)
    j = pl.program_id(1)
    n = i * _NQ + j
    s = lax.rem(n, 2)

    @pl.when(n >= 2)
    def _():
        t = n - 2
        pltpu.make_async_copy(
            scratch.at[s],
            o_hbm.at[pl.ds((t // _NQ) * _RB, _RB),
                     pl.ds(lax.rem(t, _NQ) * _QW, _QW)],
            sems.at[s],
        ).wait()

    h = h_ref[...]
    for k in range(_KSUB):
        sl = slice(k * 2048, (k + 1) * 2048)
        scratch[s, :, sl] = (
            jnp.dot(h, w_ref[:, sl], preferred_element_type=jnp.float32)
            + b_ref[:, sl]
        )

    pltpu.make_async_copy(
        scratch.at[s],
        o_hbm.at[pl.ds(i * _RB, _RB), pl.ds(j * _QW, _QW)],
        sems.at[s],
    ).start()

    @pl.when(j == _NQ - 1)
    def _():
        tail_buf[pl.ds(i * _RB, _RB), :] = (
            jnp.dot(h, wt_tail_ref[...], preferred_element_type=jnp.float32)
            + bt_ref[...]
        )

    @pl.when(n == _NS * _NQ - 1)
    def _():
        pltpu.make_async_copy(
            tail_buf, o_hbm.at[:, pl.ds(_MAIN, _TAIL)], tail_sem,
        ).start()
        t = _NS * _NQ - 2
        pltpu.make_async_copy(
            scratch.at[t % 2],
            o_hbm.at[pl.ds((t // _NQ) * _RB, _RB),
                     pl.ds((t % _NQ) * _QW, _QW)],
            sems.at[t % 2],
        ).wait()
        t = _NS * _NQ - 1
        pltpu.make_async_copy(
            scratch.at[t % 2],
            o_hbm.at[pl.ds((t // _NQ) * _RB, _RB),
                     pl.ds((t % _NQ) * _QW, _QW)],
            sems.at[t % 2],
        ).wait()
        pltpu.make_async_copy(
            tail_buf, o_hbm.at[:, pl.ds(_MAIN, _TAIL)], tail_sem,
        ).wait()


def _project(hidden, wt, bias2d, interpret=False):
    w_main = wt[:, :_MAIN]
    w_tail = wt[:, _MAIN:]
    b_main = bias2d[:, :_MAIN]
    b_tail = bias2d[:, _MAIN:]
    return pl.pallas_call(
        _proj_body,
        grid=(_NS, _NQ),
        interpret=interpret,
        in_specs=[
            pl.BlockSpec((_RB, HIDDEN), lambda i, j: (i, 0)),
            pl.BlockSpec((HIDDEN, _QW), lambda i, j: (0, j)),
            pl.BlockSpec((HIDDEN, _TAIL), lambda i, j: (0, 0)),
            pl.BlockSpec((1, _QW), lambda i, j: (0, j)),
            pl.BlockSpec((1, _TAIL), lambda i, j: (0, 0)),
        ],
        out_specs=pl.BlockSpec(memory_space=pl.ANY),
        out_shape=jax.ShapeDtypeStruct((BATCH, VOCAB), jnp.float32),
        scratch_shapes=[
            pltpu.VMEM((2, _RB, _QW), jnp.float32),
            pltpu.VMEM((BATCH, _TAIL), jnp.float32),
            pltpu.SemaphoreType.DMA((2,)),
            pltpu.SemaphoreType.DMA,
        ],
    )(hidden, w_main, w_tail, b_main, b_tail)


def kernel(input_ids, embed_table, proj_weight, proj_bias):
    ids = input_ids.astype(jnp.int32)
    hidden = _make_sc_gather()(ids, embed_table)
    return _project(hidden, proj_weight.T, proj_bias.reshape(1, VOCAB))


# speedup vs baseline: 1.0752x; 1.0752x over previous
import functools

import jax
import jax.numpy as jnp
from jax import lax
from jax.experimental import pallas as pl
from jax.experimental.pallas import tpu as pltpu
from jax.experimental.pallas import tpu_sc as plsc

VOCAB = 100000
HIDDEN = 32
BATCH = 1024


@functools.cache
def _make_sc_gather():
    info = plsc.get_sparse_core_info()
    nc, ns = info.num_cores, info.num_subcores
    b_per_w = BATCH // (nc * ns)
    mesh = plsc.VectorSubcoreMesh(core_axis_name="c", subcore_axis_name="s")

    @functools.partial(
        pl.kernel,
        mesh=mesh,
        out_type=jax.ShapeDtypeStruct((BATCH, HIDDEN), jnp.float32),
        scratch_types=[
            pltpu.VMEM((b_per_w,), jnp.int32),
            pltpu.VMEM((b_per_w, HIDDEN), jnp.float32),
            pltpu.SemaphoreType.DMA,
        ],
        compiler_params=pltpu.CompilerParams(use_tc_tiling_on_sc=False),
    )
    def _sc_gather(idx_hbm, table_hbm, out_hbm, idx_v, rows_v, sem):
        wid = lax.axis_index("s") * nc + lax.axis_index("c")
        base = wid * b_per_w
        pltpu.sync_copy(idx_hbm.at[pl.ds(base, b_per_w)], idx_v)
        pltpu.async_copy(table_hbm.at[idx_v], rows_v, sem).wait()
        pltpu.sync_copy(rows_v, out_hbm.at[pl.ds(base, b_per_w)])

    return _sc_gather


_RB = 64
_NS = BATCH // _RB
_CW = 2048
_NC_FULL = 48
_MAIN = _NC_FULL * _CW
_TAIL = VOCAB - _MAIN


def _proj_body(h_ref, w_ref, b_ref, o_hbm, scratch, sems):
    i = pl.program_id(0)
    s = lax.rem(i, 2)

    @pl.when(i >= 2)
    def _():
        pltpu.make_async_copy(
            scratch.at[s],
            o_hbm.at[pl.ds((i - 2) * _RB, _RB), :],
            sems.at[s],
        ).wait()

    h = h_ref[...]
    for k in range(_NC_FULL):
        sl = slice(k * _CW, (k + 1) * _CW)
        scratch[s, :, sl] = (
            jnp.dot(h, w_ref[:, sl], preferred_element_type=jnp.float32)
            + b_ref[:, sl]
        )
    scratch[s, :, _MAIN:VOCAB] = (
        jnp.dot(h, w_ref[:, _MAIN:VOCAB], preferred_element_type=jnp.float32)
        + b_ref[:, _MAIN:VOCAB]
    )

    pltpu.make_async_copy(
        scratch.at[s],
        o_hbm.at[pl.ds(i * _RB, _RB), :],
        sems.at[s],
    ).start()

    @pl.when(i == _NS - 1)
    def _():
        for t in (_NS - 2, _NS - 1):
            pltpu.make_async_copy(
                scratch.at[t % 2],
                o_hbm.at[pl.ds(t * _RB, _RB), :],
                sems.at[t % 2],
            ).wait()


def _project(hidden, wt, bias2d, interpret=False):
    return pl.pallas_call(
        _proj_body,
        grid=(_NS,),
        interpret=interpret,
        in_specs=[
            pl.BlockSpec((_RB, HIDDEN), lambda i: (i, 0)),
            pl.BlockSpec((HIDDEN, VOCAB), lambda i: (0, 0)),
            pl.BlockSpec((1, VOCAB), lambda i: (0, 0)),
        ],
        out_specs=pl.BlockSpec(memory_space=pl.ANY),
        out_shape=jax.ShapeDtypeStruct((BATCH, VOCAB), jnp.float32),
        scratch_shapes=[
            pltpu.VMEM((2, _RB, VOCAB), jnp.float32),
            pltpu.SemaphoreType.DMA((2,)),
        ],
        compiler_params=pltpu.CompilerParams(
            vmem_limit_bytes=100 * 1024 * 1024,
        ),
    )(hidden, wt, bias2d)


def kernel(input_ids, embed_table, proj_weight, proj_bias):
    ids = input_ids.astype(jnp.int32)
    hidden = _make_sc_gather()(ids, embed_table)
    return _project(hidden, proj_weight.T, proj_bias.reshape(1, VOCAB))


# speedup vs baseline: 1.0769x; 1.0016x over previous
import functools

import jax
import jax.numpy as jnp
from jax import lax
from jax.experimental import pallas as pl
from jax.experimental.pallas import tpu as pltpu
from jax.experimental.pallas import tpu_sc as plsc

VOCAB = 100000
HIDDEN = 32
BATCH = 1024


@functools.cache
def _make_sc_gather():
    info = plsc.get_sparse_core_info()
    nc, ns = info.num_cores, info.num_subcores
    b_per_w = BATCH // (nc * ns)
    mesh = plsc.VectorSubcoreMesh(core_axis_name="c", subcore_axis_name="s")

    @functools.partial(
        pl.kernel,
        mesh=mesh,
        out_type=jax.ShapeDtypeStruct((BATCH, HIDDEN), jnp.float32),
        scratch_types=[
            pltpu.VMEM((b_per_w,), jnp.int32),
            pltpu.VMEM((b_per_w, HIDDEN), jnp.float32),
            pltpu.SemaphoreType.DMA,
        ],
        compiler_params=pltpu.CompilerParams(use_tc_tiling_on_sc=False),
    )
    def _sc_gather(idx_hbm, table_hbm, out_hbm, idx_v, rows_v, sem):
        wid = lax.axis_index("s") * nc + lax.axis_index("c")
        base = wid * b_per_w
        pltpu.sync_copy(idx_hbm.at[pl.ds(base, b_per_w)], idx_v)
        pltpu.async_copy(table_hbm.at[idx_v], rows_v, sem).wait()
        pltpu.sync_copy(rows_v, out_hbm.at[pl.ds(base, b_per_w)])

    return _sc_gather


_RB = 64
_NS = BATCH // _RB
_CW = 2048
_NC_FULL = 48
_MAIN = _NC_FULL * _CW
_TAIL = VOCAB - _MAIN


def _proj_body(h_ref, w_ref, b_ref, o_ref):
    h = h_ref[...]
    for k in range(_NC_FULL):
        sl = slice(k * _CW, (k + 1) * _CW)
        o_ref[:, sl] = (
            jnp.dot(h, w_ref[:, sl], preferred_element_type=jnp.float32)
            + b_ref[:, sl]
        )
    o_ref[:, _MAIN:VOCAB] = (
        jnp.dot(h, w_ref[:, _MAIN:VOCAB], preferred_element_type=jnp.float32)
        + b_ref[:, _MAIN:VOCAB]
    )


def _project(hidden, wt, bias2d, interpret=False):
    return pl.pallas_call(
        _proj_body,
        grid=(_NS,),
        interpret=interpret,
        in_specs=[
            pl.BlockSpec((_RB, HIDDEN), lambda i: (i, 0)),
            pl.BlockSpec((HIDDEN, VOCAB), lambda i: (0, 0)),
            pl.BlockSpec((1, VOCAB), lambda i: (0, 0)),
        ],
        out_specs=pl.BlockSpec((_RB, VOCAB), lambda i: (i, 0)),
        out_shape=jax.ShapeDtypeStruct((BATCH, VOCAB), jnp.float32),
        compiler_params=pltpu.CompilerParams(
            vmem_limit_bytes=100 * 1024 * 1024,
        ),
    )(hidden, wt, bias2d)


def kernel(input_ids, embed_table, proj_weight, proj_bias):
    ids = input_ids.astype(jnp.int32)
    hidden = _make_sc_gather()(ids, embed_table)
    return _project(hidden, proj_weight.T, proj_bias.reshape(1, VOCAB))
